# Initial kernel scaffold; baseline (speedup 1.0000x reference)
import jax
import jax.numpy as jnp
from jax.experimental import pallas as pl


def kernel(x, table):
    raise NotImplementedError("write your pallas kernel here")


# speedup vs baseline: 1.4597x; 1.4597x over previous
import functools

import jax
import jax.numpy as jnp
from jax import lax
from jax.experimental import pallas as pl
from jax.experimental.pallas import tpu as pltpu
from jax.experimental.pallas import tpu_sc as plsc

_EMBED_DIM = 32
_SCALE = float(_EMBED_DIM ** 0.5)

_NUM_CORES = 2
_NUM_SUBCORES = 16
_NW = _NUM_CORES * _NUM_SUBCORES
_LANE = 16
_SUB = 128
_K = 8


def _sc_embed(x2, table):
    n_rows = x2.shape[0]
    rows_per_w = n_rows // _NW
    n_groups = rows_per_w // _K
    d = table.shape[1]

    mesh = plsc.VectorSubcoreMesh(core_axis_name="c", subcore_axis_name="s")

    @functools.partial(
        pl.kernel,
        mesh=mesh,
        out_type=jax.ShapeDtypeStruct((n_rows, _SUB, d), jnp.float32),
        compiler_params=pltpu.CompilerParams(use_tc_tiling_on_sc=False),
        scratch_types=[
            pltpu.VMEM((2, _K, _SUB), jnp.int32),
            pltpu.VMEM((2, _K, _SUB, d), jnp.float32),
            pltpu.SemaphoreType.DMA,
            pltpu.SemaphoreType.DMA,
            pltpu.SemaphoreType.DMA,
            pltpu.SemaphoreType.DMA,
        ],
    )
    def body(x_hbm, tab_hbm, out_hbm, idx_v, rows_v, g0, g1, o0, o1):
        wid = lax.axis_index("s") * _NUM_CORES + lax.axis_index("c")
        w_row = wid * rows_per_w
        gsem = (g0, g1)
        osem = (o0, o1)
        gh = [None, None]
        oh = [None, None]

        def fire(g, b):
            row = w_row + g * _K
            if oh[b] is not None:
                oh[b].wait()
                oh[b] = None
            pltpu.sync_copy(x_hbm.at[pl.ds(row, _K)], idx_v.at[b])
            gh[b] = [
                pltpu.async_copy(
                    tab_hbm.at[idx_v.at[b, j]], rows_v.at[b, j], gsem[b])
                for j in range(_K)
            ]

        def scale_store(g, b):
            for h in gh[b]:
                h.wait()
            gh[b] = None

            def sbody(r, carry):
                for j in range(_K):
                    v0 = rows_v[b, j, r, pl.ds(0, _LANE)]
                    rows_v[b, j, r, pl.ds(0, _LANE)] = v0 * _SCALE
                    v1 = rows_v[b, j, r, pl.ds(_LANE, _LANE)]
                    rows_v[b, j, r, pl.ds(_LANE, _LANE)] = v1 * _SCALE
                return carry

            lax.fori_loop(0, _SUB, sbody, 0)
            row = w_row + g * _K
            oh[b] = pltpu.async_copy(
                rows_v.at[b], out_hbm.at[pl.ds(row, _K)], osem[b])

        fire(0, 0)
        for g in range(n_groups):
            if g + 1 < n_groups:
                fire(g + 1, (g + 1) % 2)
            scale_store(g, g % 2)
        for b in range(2):
            if oh[b] is not None:
                oh[b].wait()

    return body(x2, table)


def kernel(x, table):
    batch, hist = x.shape
    total = batch * hist
    x2 = x.astype(jnp.int32).reshape(total // _SUB, _SUB)
    out = _sc_embed(x2, table)
    return out.reshape(batch, hist, table.shape[1])
